# baseline (speedup 1.0000x reference)
import functools

import jax
import jax.numpy as jnp
from jax import lax
from jax.experimental import pallas as pl
from jax.experimental.pallas import tpu as pltpu
from jax.experimental.pallas import tpu_sc as plsc

LANES = 16
CHUNK = 64
SUP = 16
NC = 2
NS = 16
NW = NC * NS


def _proj_body(x_ref, w_ref, att2_ref, h_ref, a2_ref):
    h = jnp.dot(x_ref[...], w_ref[...], preferred_element_type=jnp.float32)
    h_ref[...] = h
    a2_ref[...] = jnp.dot(h, att2_ref[...], preferred_element_type=jnp.float32)


def _final_body(o_ref, d_ref, b_ref, out_ref):
    o = o_ref[0] + o_ref[1]
    d = d_ref[0] + d_ref[1]
    v = o / (d + 1e-16) + b_ref[...]
    out_ref[...] = jnp.where(v > 0, v, jnp.exp(jnp.minimum(v, 0.0)) - 1.0)


def _make_sc_kernel(n_nodes, d_out, n_edges, sup_per_tile, n_pad):
    per_tile = sup_per_tile * SUP * CHUNK
    row_blk = (n_nodes // NS) // 8 * 8
    row_last = n_nodes - (NS - 1) * row_blk

    mesh = plsc.VectorSubcoreMesh(core_axis_name="c", subcore_axis_name="s")

    @functools.partial(
        pl.kernel,
        out_type=(
            jax.ShapeDtypeStruct((NC, n_nodes, d_out), jnp.float32),
            jax.ShapeDtypeStruct((NC * n_pad,), jnp.float32),
        ),
        mesh=mesh,
        compiler_params=pltpu.CompilerParams(needs_layout_passes=False),
        scratch_types=[
            pltpu.VMEM((2, SUP, CHUNK), jnp.int32),
            pltpu.VMEM((2, SUP, CHUNK), jnp.int32),
            pltpu.VMEM((2, CHUNK), jnp.float32),
            pltpu.VMEM((2, CHUNK), jnp.float32),
            pltpu.VMEM((2, CHUNK), jnp.float32),
            pltpu.VMEM((2, CHUNK, d_out), jnp.float32),
            pltpu.VMEM((CHUNK, d_out), jnp.float32),
            pltpu.VMEM_SHARED((n_nodes + CHUNK, d_out), jnp.float32),
            pltpu.VMEM_SHARED((n_pad,), jnp.float32),
            pltpu.SemaphoreType.DMA,
            pltpu.SemaphoreType.DMA,
            pltpu.SemaphoreType.DMA,
            pltpu.SemaphoreType.DMA,
            pltpu.SemaphoreType.DMA,
            pltpu.SemaphoreType.DMA,
            pltpu.SemaphoreType.DMA,
            pltpu.SemaphoreType.DMA,
            pltpu.SemaphoreType.DMA,
            pltpu.SemaphoreType.DMA,
        ],
    )
    def sc_kernel(h_hbm, asrc_hbm, adst_hbm, ei_hbm,
                  outp_hbm, denp_hbm,
                  sidx_v, didx_v, av_v, bv_v, ex_v, rows_v, srows_v,
                  out_sh, den_sh, semidx,
                  semr0, semr1, sema0, sema1, semb0, semb1,
                  semd0, semd1, semo):
        semr = (semr0, semr1)
        sema = (sema0, sema1)
        semb = (semb0, semb1)
        semd = (semd0, semd1)
        cid = lax.axis_index("c")
        sid = lax.axis_index("s")
        wid = cid * NS + sid

        zero_scope = jax.named_scope("sc_zero")
        zero_scope.__enter__()

        def _zrow(r, _):
            for k in range(d_out // LANES):
                srows_v[r, pl.ds(k * LANES, LANES)] = jnp.zeros(
                    (LANES,), jnp.float32)
            return 0
        lax.fori_loop(0, CHUNK, _zrow, 0)
        for g in range(CHUNK // LANES):
            ex_v[0, pl.ds(g * LANES, LANES)] = jnp.zeros(
                (LANES,), jnp.float32)

        def _zero_rows(start, count):
            off = 0
            while off < count:
                blk = min(CHUNK, count - off)
                pltpu.sync_copy(srows_v.at[pl.ds(0, blk)],
                                out_sh.at[pl.ds(start + off, blk)])
                off += blk

        @pl.when(sid < NS - 1)
        def _():
            _zero_rows(sid * row_blk, row_blk)

        @pl.when(sid == NS - 1)
        def _():
            _zero_rows((NS - 1) * row_blk, row_last)

        den_per_tile = n_pad // NS
        den_base = sid * den_per_tile
        for off2 in range(0, den_per_tile, CHUNK):
            pltpu.sync_copy(ex_v.at[0],
                            den_sh.at[pl.ds(den_base + off2, CHUNK)])

        plsc.subcore_barrier()
        zero_scope.__exit__(None, None, None)

        base_edge = wid * per_tile
        n_blocks = sup_per_tile
        n_chunks = n_blocks * SUP

        def _idx_rows(t):
            q = (t // SUP) % 2
            r = t % SUP
            return sidx_v.at[q, r], didx_v.at[q, r]

        def _issue_gathers(t, p):
            s_row, d_row = _idx_rows(t)
            pltpu.async_copy(h_hbm.at[s_row], rows_v.at[p], semr[p])
            pltpu.async_copy(asrc_hbm.at[s_row], av_v.at[p], sema[p])
            pltpu.async_copy(adst_hbm.at[d_row], bv_v.at[p], semb[p])

        def _wait_den_scat(p):
            s_row, d_row = _idx_rows(0)
            pltpu.make_async_copy(ex_v.at[p], den_sh.at[d_row],
                                  semd[p]).wait()

        def _wait_out_scat():
            s_row, d_row = _idx_rows(0)
            pltpu.make_async_copy(srows_v, out_sh.at[d_row], semo).wait()

        def _process(t, p):
            s_row, d_row = _idx_rows(t)
            @pl.when(t > 1)
            def _():
                _wait_den_scat(p)
            pltpu.make_async_copy(asrc_hbm.at[s_row], av_v.at[p],
                                  sema[p]).wait()
            pltpu.make_async_copy(adst_hbm.at[d_row], bv_v.at[p],
                                  semb[p]).wait()

            def g_body(g, _):
                e = (av_v[p, pl.ds(g * LANES, LANES)]
                     + bv_v[p, pl.ds(g * LANES, LANES)])
                e = jnp.where(e >= 0.0, e, 0.2 * e)
                ex = jnp.exp(e)
                gid = (base_edge + t * CHUNK + g * LANES
                       + lax.iota(jnp.int32, 16))
                ex = jnp.where(gid < n_edges, ex, 0.0)
                ex_v[p, pl.ds(g * LANES, LANES)] = ex
                return 0
            lax.fori_loop(0, CHUNK // LANES, g_body, 0)

            pltpu.async_copy(ex_v.at[p], den_sh.at[d_row], semd[p],
                             add=True)

            pltpu.make_async_copy(h_hbm.at[s_row], rows_v.at[p],
                                  semr[p]).wait()

            @pl.when(t > 0)
            def _():
                _wait_out_scat()

            def m_body(g, _):
                for j in range(LANES):
                    rr = g * LANES + j
                    sp = plsc.load_gather(
                        ex_v.at[p], [jnp.full((LANES,), rr, jnp.int32)])
                    for k in range(d_out // LANES):
                        srows_v[rr, pl.ds(k * LANES, LANES)] = (
                            rows_v[p, rr, pl.ds(k * LANES, LANES)] * sp)
                return 0
            lax.fori_loop(0, CHUNK // LANES, m_body, 0)

            pltpu.async_copy(srows_v, out_sh.at[d_row], semo, add=True)

            tn = t + 2

            @pl.when(tn < n_chunks)
            def _():
                @pl.when(tn % SUP == 0)
                def _():
                    _wait_idx_block(tn // SUP)
                _issue_gathers(tn, p)

        def _wait_idx_block(b):
            qb = b % 2
            pltpu.make_async_copy(ei_hbm.at[0, wid, b], sidx_v.at[qb],
                                  semidx).wait()
            pltpu.make_async_copy(ei_hbm.at[1, wid, b], didx_v.at[qb],
                                  semidx).wait()

        with jax.named_scope("sc_prologue"):
            pltpu.sync_copy(ei_hbm.at[0, wid, 0], sidx_v.at[0])
            pltpu.sync_copy(ei_hbm.at[1, wid, 0], didx_v.at[0])
            _issue_gathers(0, 0)
            _issue_gathers(1, 1)

        def pipe_body(u, _):
            ta = 2 * u
            @pl.when(ta % SUP == 0)
            def _():
                b = ta // SUP

                @pl.when(b + 1 < n_blocks)
                def _():
                    qn = (b + 1) % 2
                    pltpu.async_copy(ei_hbm.at[0, wid, b + 1], sidx_v.at[qn],
                                     semidx)
                    pltpu.async_copy(ei_hbm.at[1, wid, b + 1], didx_v.at[qn],
                                     semidx)

            _process(ta, 0)
            _process(ta + 1, 1)
            return 0

        with jax.named_scope("sc_mainloop"):
            lax.fori_loop(0, n_chunks // 2, pipe_body, 0)

        with jax.named_scope("sc_epilogue"):
            _wait_den_scat(0)
            _wait_den_scat(1)
            _wait_out_scat()

            plsc.subcore_barrier()

        with jax.named_scope("sc_drain"):
            @pl.when(sid < NS - 1)
            def _():
                pltpu.sync_copy(
                    out_sh.at[pl.ds(sid * row_blk, row_blk)],
                    outp_hbm.at[cid, pl.ds(sid * row_blk, row_blk)])

            @pl.when(sid == NS - 1)
            def _():
                pltpu.sync_copy(
                    out_sh.at[pl.ds((NS - 1) * row_blk, row_last)],
                    outp_hbm.at[cid, pl.ds((NS - 1) * row_blk, row_last)])

            @pl.when(sid == 0)
            def _():
                pltpu.sync_copy(den_sh,
                                denp_hbm.at[pl.ds(cid * n_pad, n_pad)])

    return sc_kernel


def kernel(input, edge_index, W, att_src, att_dst, bias):
    n, d_in = input.shape
    d_out = W.shape[1]
    n_edges = edge_index.shape[1]

    bn = 2000
    att2 = jnp.stack([att_src, att_dst], axis=1)
    h, a2 = pl.pallas_call(
        _proj_body,
        grid=(n // bn,),
        in_specs=[
            pl.BlockSpec((bn, d_in), lambda i: (i, 0)),
            pl.BlockSpec((d_in, d_out), lambda i: (0, 0)),
            pl.BlockSpec((d_out, 2), lambda i: (0, 0)),
        ],
        out_specs=[
            pl.BlockSpec((bn, d_out), lambda i: (i, 0)),
            pl.BlockSpec((bn, 2), lambda i: (i, 0)),
        ],
        out_shape=[
            jax.ShapeDtypeStruct((n, d_out), jnp.float32),
            jax.ShapeDtypeStruct((n, 2), jnp.float32),
        ],
    )(input, W, att2)
    asrc = a2[:, 0]
    adst = a2[:, 1]

    sup_per_tile = -(-n_edges // (NW * SUP * CHUNK))
    e_pad = NW * sup_per_tile * SUP * CHUNK
    npad_e = e_pad - n_edges
    pad_iota = lax.iota(jnp.int32, npad_e)
    pad_block = jnp.stack([pad_iota % n, n + (pad_iota % CHUNK)])
    ei = jnp.concatenate([edge_index.astype(jnp.int32), pad_block], axis=1)
    ei = ei.reshape(2, NW, sup_per_tile, SUP, CHUNK)

    n_pad = -(-n // 1024) * 1024
    sc = _make_sc_kernel(n, d_out, n_edges, sup_per_tile, n_pad)
    outp, denp = sc(h, asrc, adst, ei)

    denp3 = denp.reshape(NC, n_pad)[:, :n].reshape(NC, n, 1)
    bias2 = bias.reshape(1, d_out)
    out = pl.pallas_call(
        _final_body,
        grid=(n // bn,),
        in_specs=[
            pl.BlockSpec((NC, bn, d_out), lambda i: (0, i, 0)),
            pl.BlockSpec((NC, bn, 1), lambda i: (0, i, 0)),
            pl.BlockSpec((1, d_out), lambda i: (0, 0)),
        ],
        out_specs=pl.BlockSpec((bn, d_out), lambda i: (i, 0)),
        out_shape=jax.ShapeDtypeStruct((n, d_out), jnp.float32),
    )(outp, denp3, bias2)
    return out
